# baseline (speedup 1.0000x reference)
import functools

import jax
import jax.numpy as jnp
from jax import lax
from jax.experimental import pallas as pl
from jax.experimental.pallas import tpu as pltpu
from jax.experimental.pallas import tpu_sc as plsc

_GDN = lax.GatherDimensionNumbers(
    offset_dims=(), collapsed_slice_dims=(0,), start_index_map=(0,))


def _shuffle(v, perm):
  return lax.gather(v, perm[:, None], _GDN, slice_sizes=(1,),
                    mode=lax.GatherScatterMode.PROMISE_IN_BOUNDS)


BATCH = 16384
D = 128
L = 16
NC, NS = 2, 16
NW = NC * NS
BPW = BATCH // NW
CH = 64
NCH = BPW // CH
G = CH // L


def _build():
  mesh = plsc.VectorSubcoreMesh(core_axis_name="c", subcore_axis_name="s")

  @functools.partial(
      pl.kernel,
      out_type=jax.ShapeDtypeStruct((BATCH,), jnp.float32),
      mesh=mesh,
      scratch_types=[
          pltpu.VMEM((BPW,), jnp.int32),
          pltpu.VMEM((BPW,), jnp.int32),
          pltpu.VMEM((2, CH, D), jnp.float32),
          pltpu.VMEM((2, CH, D), jnp.float32),
          pltpu.VMEM((2 * D,), jnp.float32),
          pltpu.VMEM((L,), jnp.float32),
          pltpu.VMEM((BPW,), jnp.float32),
          pltpu.SemaphoreType.DMA,
          pltpu.SemaphoreType.DMA,
          pltpu.SemaphoreType.DMA,
          pltpu.SemaphoreType.DMA,
      ],
  )
  def mfnet(uid_h, iid_h, uemb_h, iemb_h, w_h, b_h, out_h,
            uidx_v, iidx_v, urows_v, irows_v, w_v, b_v, z_v,
            sem_u0, sem_u1, sem_i0, sem_i1):
    sem_u = (sem_u0, sem_u1)
    sem_i = (sem_i0, sem_i1)
    wid = lax.axis_index("s") * NC + lax.axis_index("c")
    base = wid * BPW
    pltpu.sync_copy(uid_h.at[pl.ds(base, BPW)], uidx_v)
    pltpu.sync_copy(iid_h.at[pl.ds(base, BPW)], iidx_v)
    pltpu.sync_copy(w_h, w_v)
    pltpu.sync_copy(b_h, b_v)

    wu = [w_v[pl.ds(L * t, L)] for t in range(D // L)]
    wi = [w_v[pl.ds(D + L * t, L)] for t in range(D // L)]
    bv = b_v[...]
    lanes = lax.iota(jnp.int32, L)
    perms = [lanes ^ (1 << k) for k in range(4)]
    masks = [((lanes >> k) & 1) == 0 for k in range(4)]

    def comb(k, a, b):
      return jnp.where(masks[k], a + _shuffle(a, perms[k]),
                       b + _shuffle(b, perms[k]))

    def issue(c):
      buf = c % 2
      cu = pltpu.async_copy(
          uemb_h.at[uidx_v.at[pl.ds(c * CH, CH)]], urows_v.at[buf],
          sem_u[buf])
      ci = pltpu.async_copy(
          iemb_h.at[iidx_v.at[pl.ds(c * CH, CH)]], irows_v.at[buf],
          sem_i[buf])
      return cu, ci

    inflight = {0: issue(0)}
    for c in range(NCH):
      if c + 1 < NCH:
        inflight[c + 1] = issue(c + 1)
      cu, ci = inflight.pop(c)
      cu.wait()
      ci.wait()
      ubuf = urows_v.at[c % 2]
      ibuf = irows_v.at[c % 2]

      def group_body(g, _):
        r0 = g * L
        accs = []
        for rr in range(L):
          r = r0 + rr
          accu = ubuf[r, pl.ds(0, L)] * wu[0]
          acci = ibuf[r, pl.ds(0, L)] * wi[0]
          for t in range(1, D // L):
            accu = accu + ubuf[r, pl.ds(L * t, L)] * wu[t]
            acci = acci + ibuf[r, pl.ds(L * t, L)] * wi[t]
          accs.append(accu + acci)
        for k in range(4):
          accs = [comb(k, accs[2 * j], accs[2 * j + 1])
                  for j in range(len(accs) // 2)]
        zvec = accs[0] + bv
        z_v[pl.ds(c * CH + r0, L)] = 1.0 / (1.0 + jnp.exp(-zvec))
        return 0

      lax.fori_loop(0, G, group_body, 0)

    pltpu.sync_copy(z_v, out_h.at[pl.ds(base, BPW)])

  return mfnet


_MFNET = _build()


def kernel(user_id, item_id, user_emb, item_emb, W, b):
  w_flat = W.reshape(2 * D)
  b_vec = jnp.broadcast_to(b, (L,))
  return _MFNET(user_id.astype(jnp.int32), item_id.astype(jnp.int32),
                user_emb, item_emb, w_flat, b_vec)


# speedup vs baseline: 1.1686x; 1.1686x over previous
import functools

import jax
import jax.numpy as jnp
from jax import lax
from jax.experimental import pallas as pl
from jax.experimental.pallas import tpu as pltpu
from jax.experimental.pallas import tpu_sc as plsc

_GDN = lax.GatherDimensionNumbers(
    offset_dims=(), collapsed_slice_dims=(0,), start_index_map=(0,))


def _shuffle(v, perm):
  return lax.gather(v, perm[:, None], _GDN, slice_sizes=(1,),
                    mode=lax.GatherScatterMode.PROMISE_IN_BOUNDS)


BATCH = 16384
D = 128
L = 16
NC, NS = 2, 16
NW = NC * NS
BPW = BATCH // NW
CH = 128
NCH = BPW // CH
G = CH // L


def _build():
  mesh = plsc.VectorSubcoreMesh(core_axis_name="c", subcore_axis_name="s")

  @functools.partial(
      pl.kernel,
      out_type=jax.ShapeDtypeStruct((BATCH,), jnp.float32),
      mesh=mesh,
      scratch_types=[
          pltpu.VMEM((BPW,), jnp.int32),
          pltpu.VMEM((BPW,), jnp.int32),
          pltpu.VMEM((2, CH, D), jnp.float32),
          pltpu.VMEM((2, CH, D), jnp.float32),
          pltpu.VMEM((2 * D,), jnp.float32),
          pltpu.VMEM((L,), jnp.float32),
          pltpu.VMEM((BPW,), jnp.float32),
          pltpu.SemaphoreType.DMA,
          pltpu.SemaphoreType.DMA,
          pltpu.SemaphoreType.DMA,
          pltpu.SemaphoreType.DMA,
          pltpu.SemaphoreType.DMA,
      ],
  )
  def mfnet(uid_h, iid_h, uemb_h, iemb_h, w_h, b_h, out_h,
            uidx_v, iidx_v, urows_v, irows_v, w_v, b_v, z_v,
            sem_u0, sem_u1, sem_i0, sem_i1, sem_o):
    sem_u = (sem_u0, sem_u1)
    sem_i = (sem_i0, sem_i1)
    wid = lax.axis_index("s") * NC + lax.axis_index("c")
    base = wid * BPW
    cpu = pltpu.async_copy(uid_h.at[pl.ds(base, BPW)], uidx_v, sem_u0)
    cpi = pltpu.async_copy(iid_h.at[pl.ds(base, BPW)], iidx_v, sem_i0)
    cpu.wait()
    cpi.wait()

    def issue(c):
      buf = c % 2
      cu = pltpu.async_copy(
          uemb_h.at[uidx_v.at[pl.ds(c * CH, CH)]], urows_v.at[buf],
          sem_u[buf])
      ci = pltpu.async_copy(
          iemb_h.at[iidx_v.at[pl.ds(c * CH, CH)]], irows_v.at[buf],
          sem_i[buf])
      return cu, ci

    inflight = {0: issue(0), 1: issue(1)}
    pltpu.sync_copy(w_h, w_v)
    pltpu.sync_copy(b_h, b_v)

    wu = [w_v[pl.ds(L * t, L)] for t in range(D // L)]
    wi = [w_v[pl.ds(D + L * t, L)] for t in range(D // L)]
    bv = b_v[...]
    lanes = lax.iota(jnp.int32, L)
    perms = [lanes ^ (1 << k) for k in range(4)]
    masks = [((lanes >> k) & 1) == 0 for k in range(4)]

    def comb(k, a, b):
      return jnp.where(masks[k], a + _shuffle(a, perms[k]),
                       b + _shuffle(b, perms[k]))

    out_cps = []
    for c in range(NCH):
      if c + 1 < NCH and c + 1 not in inflight:
        inflight[c + 1] = issue(c + 1)
      cu, ci = inflight.pop(c)
      cu.wait()
      ci.wait()
      ubuf = urows_v.at[c % 2]
      ibuf = irows_v.at[c % 2]

      def group_body(g, _):
        r0 = g * L
        accs = []
        for rr in range(L):
          r = r0 + rr
          accu = ubuf[r, pl.ds(0, L)] * wu[0]
          acci = ibuf[r, pl.ds(0, L)] * wi[0]
          for t in range(1, D // L):
            accu = accu + ubuf[r, pl.ds(L * t, L)] * wu[t]
            acci = acci + ibuf[r, pl.ds(L * t, L)] * wi[t]
          accs.append(accu + acci)
        for k in range(4):
          accs = [comb(k, accs[2 * j], accs[2 * j + 1])
                  for j in range(len(accs) // 2)]
        zvec = accs[0] + bv
        z_v[pl.ds(c * CH + r0, L)] = 1.0 / (1.0 + jnp.exp(-zvec))
        return 0

      lax.fori_loop(0, G, group_body, 0)
      out_cps.append(pltpu.async_copy(
          z_v.at[pl.ds(c * CH, CH)], out_h.at[pl.ds(base + c * CH, CH)],
          sem_o))

    for cp in out_cps:
      cp.wait()

  return mfnet


_MFNET = _build()


def kernel(user_id, item_id, user_emb, item_emb, W, b):
  w_flat = W.reshape(2 * D)
  b_vec = jnp.broadcast_to(b, (L,))
  return _MFNET(user_id.astype(jnp.int32), item_id.astype(jnp.int32),
                user_emb, item_emb, w_flat, b_vec)


# speedup vs baseline: 1.1952x; 1.0227x over previous
import functools

import jax
import jax.numpy as jnp
from jax import lax
from jax.experimental import pallas as pl
from jax.experimental.pallas import tpu as pltpu
from jax.experimental.pallas import tpu_sc as plsc

_GDN = lax.GatherDimensionNumbers(
    offset_dims=(), collapsed_slice_dims=(0,), start_index_map=(0,))


def _shuffle(v, perm):
  return lax.gather(v, perm[:, None], _GDN, slice_sizes=(1,),
                    mode=lax.GatherScatterMode.PROMISE_IN_BOUNDS)


BATCH = 16384
D = 128
L = 16
NC, NS = 2, 16
NW = NC * NS
BPW = BATCH // NW
CHS = (240, 240, 32)
OFFS = (0, 240, 480)
CHMAX = CHS[0]
NCH = len(CHS)


def _build():
  mesh = plsc.VectorSubcoreMesh(core_axis_name="c", subcore_axis_name="s")

  @functools.partial(
      pl.kernel,
      out_type=jax.ShapeDtypeStruct((BATCH,), jnp.float32),
      mesh=mesh,
      scratch_types=[
          pltpu.VMEM((BPW,), jnp.int32),
          pltpu.VMEM((BPW,), jnp.int32),
          pltpu.VMEM((2, CHMAX, D), jnp.float32),
          pltpu.VMEM((2, CHMAX, D), jnp.float32),
          pltpu.VMEM((2 * D,), jnp.float32),
          pltpu.VMEM((L,), jnp.float32),
          pltpu.VMEM((BPW,), jnp.float32),
          pltpu.SemaphoreType.DMA,
          pltpu.SemaphoreType.DMA,
          pltpu.SemaphoreType.DMA,
          pltpu.SemaphoreType.DMA,
          pltpu.SemaphoreType.DMA,
      ],
  )
  def mfnet(uid_h, iid_h, uemb_h, iemb_h, w_h, b_h, out_h,
            uidx_v, iidx_v, urows_v, irows_v, w_v, b_v, z_v,
            sem_u0, sem_u1, sem_i0, sem_i1, sem_o):
    sem_u = (sem_u0, sem_u1)
    sem_i = (sem_i0, sem_i1)
    wid = lax.axis_index("s") * NC + lax.axis_index("c")
    base = wid * BPW
    cpu = pltpu.async_copy(uid_h.at[pl.ds(base, BPW)], uidx_v, sem_u0)
    cpi = pltpu.async_copy(iid_h.at[pl.ds(base, BPW)], iidx_v, sem_i0)
    cpu.wait()
    cpi.wait()

    def issue(c):
      buf = c % 2
      n = CHS[c]
      cu = pltpu.async_copy(
          uemb_h.at[uidx_v.at[pl.ds(OFFS[c], n)]],
          urows_v.at[buf, pl.ds(0, n)], sem_u[buf])
      ci = pltpu.async_copy(
          iemb_h.at[iidx_v.at[pl.ds(OFFS[c], n)]],
          irows_v.at[buf, pl.ds(0, n)], sem_i[buf])
      return cu, ci

    inflight = {0: issue(0), 1: issue(1)}
    pltpu.sync_copy(w_h, w_v)
    pltpu.sync_copy(b_h, b_v)

    wu = [w_v[pl.ds(L * t, L)] for t in range(D // L)]
    wi = [w_v[pl.ds(D + L * t, L)] for t in range(D // L)]
    bv = b_v[...]
    lanes = lax.iota(jnp.int32, L)
    perms = [lanes ^ (1 << k) for k in range(4)]
    masks = [((lanes >> k) & 1) == 0 for k in range(4)]

    def comb(k, a, b):
      return jnp.where(masks[k], a + _shuffle(a, perms[k]),
                       b + _shuffle(b, perms[k]))

    out_cps = []
    for c in range(NCH):
      if c + 1 < NCH and c + 1 not in inflight:
        inflight[c + 1] = issue(c + 1)
      cu, ci = inflight.pop(c)
      cu.wait()
      ci.wait()
      ubuf = urows_v.at[c % 2]
      ibuf = irows_v.at[c % 2]

      def group_body(g, _):
        r0 = g * L
        accs = []
        for rr in range(L):
          r = r0 + rr
          accu = ubuf[r, pl.ds(0, L)] * wu[0]
          acci = ibuf[r, pl.ds(0, L)] * wi[0]
          for t in range(1, D // L):
            accu = accu + ubuf[r, pl.ds(L * t, L)] * wu[t]
            acci = acci + ibuf[r, pl.ds(L * t, L)] * wi[t]
          accs.append(accu + acci)
        for k in range(4):
          accs = [comb(k, accs[2 * j], accs[2 * j + 1])
                  for j in range(len(accs) // 2)]
        zvec = accs[0] + bv
        z_v[pl.ds(OFFS[c] + r0, L)] = 1.0 / (1.0 + jnp.exp(-zvec))
        return 0

      lax.fori_loop(0, CHS[c] // L, group_body, 0)
      out_cps.append(pltpu.async_copy(
          z_v.at[pl.ds(OFFS[c], CHS[c])],
          out_h.at[pl.ds(base + OFFS[c], CHS[c])], sem_o))

    for cp in out_cps:
      cp.wait()

  return mfnet


_MFNET = _build()


def kernel(user_id, item_id, user_emb, item_emb, W, b):
  w_flat = W.reshape(2 * D)
  b_vec = jnp.broadcast_to(b, (L,))
  return _MFNET(user_id.astype(jnp.int32), item_id.astype(jnp.int32),
                user_emb, item_emb, w_flat, b_vec)
